# baseline (speedup 1.0000x reference)
import jax
import jax.numpy as jnp
from jax import lax
from jax.experimental import pallas as pl
from jax.experimental.pallas import tpu as pltpu
from jax.experimental.pallas import tpu_sc as plsc

N_NODES = 10000
E_EDGES = 320000
D = 128
N_CLS = 47

NC = 2
NS = 16
NW = NC * NS
CHUNK = 128
NCHUNK = 80
HALF = NCHUNK // 2
EPT = NCHUNK * CHUNK
E_PAD = NW * EPT
ROWS_PT = 640
N_PAD = NS * ROWS_PT
DUMMY_DST = N_NODES + 16
DEGW = 128


_MESH = plsc.VectorSubcoreMesh(core_axis_name="c", subcore_axis_name="s")


def _make_sc_agg():
    scratch = (
        pltpu.VMEM((NCHUNK, CHUNK), jnp.int32),
        pltpu.VMEM((NCHUNK, CHUNK), jnp.int32),
        pltpu.VMEM((CHUNK, D), jnp.float32),
        pltpu.VMEM_SHARED((N_PAD, D), jnp.float32),
    )

    def fn(h_hbm, src_hbm, dst_hbm, zacc_hbm,
           acc_out, src_v, dst_v, buf, acc_sh):
        c = lax.axis_index("c")
        s = lax.axis_index("s")
        t = c * NS + s
        base = s * ROWS_PT
        pltpu.sync_copy(zacc_hbm, acc_sh.at[pl.ds(base, ROWS_PT)])
        pltpu.sync_copy(src_hbm.at[t], src_v)
        pltpu.sync_copy(dst_hbm.at[t], dst_v)
        plsc.subcore_barrier()

        def step(j, carry):
            pltpu.sync_copy(h_hbm.at[src_v.at[j]], buf)
            pltpu.sync_copy(buf, acc_sh.at[dst_v.at[j]], add=True)
            return carry

        lax.fori_loop(0, NCHUNK, step, 0)
        plsc.subcore_barrier()
        pltpu.sync_copy(acc_sh.at[pl.ds(base, ROWS_PT)],
                        acc_out.at[c, pl.ds(base, ROWS_PT)])

    return pl.kernel(fn,
                     out_type=jax.ShapeDtypeStruct((NC, N_PAD, D),
                                                   jnp.float32),
                     mesh=_MESH, scratch_types=scratch)


def _make_sc_deg():
    scratch = (
        pltpu.VMEM((NCHUNK, CHUNK), jnp.int32),
        pltpu.VMEM((CHUNK, DEGW), jnp.float32),
        pltpu.VMEM_SHARED((N_PAD, DEGW), jnp.float32),
    )

    def fn(dst_hbm, zdeg_hbm, ones_hbm, deg_out, dst_v, ones_v, deg_sh):
        c = lax.axis_index("c")
        s = lax.axis_index("s")
        t = c * NS + s
        base = s * ROWS_PT
        pltpu.sync_copy(zdeg_hbm, deg_sh.at[pl.ds(base, ROWS_PT)])
        pltpu.sync_copy(ones_hbm, ones_v)
        pltpu.sync_copy(dst_hbm.at[t], dst_v)
        plsc.subcore_barrier()

        def step(j, carry):
            pltpu.sync_copy(ones_v, deg_sh.at[dst_v.at[j]], add=True)
            return carry

        lax.fori_loop(0, NCHUNK, step, 0)
        plsc.subcore_barrier()
        pltpu.sync_copy(deg_sh.at[pl.ds(base, ROWS_PT)],
                        deg_out.at[c, pl.ds(base, ROWS_PT)])

    return pl.kernel(fn,
                     out_type=jax.ShapeDtypeStruct((NC, N_PAD, DEGW),
                                                   jnp.float32),
                     mesh=_MESH, scratch_types=scratch)


_sc_agg = _make_sc_agg()
_sc_deg = _make_sc_deg()

_TCB = 1000


def _make_tc_layer(ln_relu: bool):
    def body(h_ref, a0_ref, a1_ref, d0_ref, d1_ref, ws_ref, wn_ref,
             b_ref, g_ref, be_ref, o_ref):
        agg = a0_ref[0] + a1_ref[0]
        deg = d0_ref[0][:, :1] + d1_ref[0][:, :1]
        mean = agg / jnp.maximum(deg, 1.0)
        z = (jnp.dot(h_ref[...], ws_ref[...],
                     preferred_element_type=jnp.float32)
             + jnp.dot(mean, wn_ref[...],
                       preferred_element_type=jnp.float32)
             + b_ref[...])
        if ln_relu:
            mu = jnp.mean(z, axis=-1, keepdims=True)
            var = jnp.mean((z - mu) ** 2, axis=-1, keepdims=True)
            z = (z - mu) * lax.rsqrt(var + 1e-5) * g_ref[...] + be_ref[...]
            z = jnp.maximum(z, 0.0)
        o_ref[...] = z

    grid = (N_NODES // _TCB,)
    full = pl.BlockSpec((1, D), lambda i: (0, 0))
    return pl.pallas_call(
        body,
        grid=grid,
        in_specs=[
            pl.BlockSpec((_TCB, D), lambda i: (i, 0)),
            pl.BlockSpec((1, _TCB, D), lambda i: (0, i, 0)),
            pl.BlockSpec((1, _TCB, D), lambda i: (1, i, 0)),
            pl.BlockSpec((1, _TCB, DEGW), lambda i: (0, i, 0)),
            pl.BlockSpec((1, _TCB, DEGW), lambda i: (1, i, 0)),
            pl.BlockSpec((D, D), lambda i: (0, 0)),
            pl.BlockSpec((D, D), lambda i: (0, 0)),
            full, full, full,
        ],
        out_specs=pl.BlockSpec((_TCB, D), lambda i: (i, 0)),
        out_shape=jax.ShapeDtypeStruct((N_NODES, D), jnp.float32),
    )


_tc_layer_lnrelu = _make_tc_layer(True)
_tc_layer_plain = _make_tc_layer(False)


def kernel(x, edge_index, W_self0, W_neigh0, b0, g0, be0,
           W_self1, W_neigh1, b1, g1, be1, W_self2, W_neigh2, b2):
    pad = E_PAD - E_EDGES
    src = jnp.concatenate(
        [edge_index[0], jnp.zeros((pad,), jnp.int32)]
    ).reshape(NW, NCHUNK, CHUNK)
    dst = jnp.concatenate(
        [edge_index[1], jnp.full((pad,), DUMMY_DST, jnp.int32)]
    ).reshape(NW, NCHUNK, CHUNK)

    zacc = jnp.zeros((ROWS_PT, D), jnp.float32)
    zdeg = jnp.zeros((ROWS_PT, DEGW), jnp.float32)
    ones_c = jnp.ones((CHUNK, DEGW), jnp.float32)

    def pad_w(w):
        return jnp.pad(w, ((0, 0), (0, D - N_CLS)))

    ws2 = pad_w(W_self2)
    wn2 = pad_w(W_neigh2)
    b2p = jnp.pad(b2, (0, D - N_CLS))

    r2 = lambda v: v.reshape(1, D)

    deg = _sc_deg(dst, zdeg, ones_c)
    acc0 = _sc_agg(x, src, dst, zacc)
    h1 = _tc_layer_lnrelu(x, acc0, acc0, deg, deg, W_self0, W_neigh0,
                          r2(b0), r2(g0), r2(be0))
    acc1 = _sc_agg(h1, src, dst, zacc)
    h2 = _tc_layer_lnrelu(h1, acc1, acc1, deg, deg, W_self1, W_neigh1,
                          r2(b1), r2(g1), r2(be1))
    acc2 = _sc_agg(h2, src, dst, zacc)
    out = _tc_layer_plain(h2, acc2, acc2, deg, deg, ws2, wn2,
                          r2(b2p), r2(b2p), r2(b2p))
    return out[:, :N_CLS]


# speedup vs baseline: 1.7406x; 1.7406x over previous
import jax
import jax.numpy as jnp
from jax import lax
from jax.experimental import pallas as pl
from jax.experimental.pallas import tpu as pltpu
from jax.experimental.pallas import tpu_sc as plsc

N_NODES = 10000
E_EDGES = 320000
D = 128
N_CLS = 47

NC = 2
NS = 16
NW = NC * NS
CHUNK = 128
NCHUNK0 = 37
NCHUNK1 = 120
NCHUNK = max(NCHUNK0, NCHUNK1)
EPT0 = NCHUNK0 * CHUNK
EPT1 = NCHUNK1 * CHUNK
E_PAD = NS * (EPT0 + EPT1)
ROWS_PT = 640
N_PAD = NS * ROWS_PT
DUMMY_DST = N_NODES + 16
DEGW = 128


_MESH = plsc.VectorSubcoreMesh(core_axis_name="c", subcore_axis_name="s")


def _make_sc_agg():
    scratch = (
        pltpu.VMEM((NCHUNK, CHUNK), jnp.int32),
        pltpu.VMEM((NCHUNK, CHUNK), jnp.int32),
        pltpu.VMEM((CHUNK, D), jnp.float32),
        pltpu.VMEM_SHARED((N_PAD, D), jnp.float32),
    )

    def fn(h_hbm, src_hbm, dst_hbm, zacc_hbm,
           acc_out, src_v, dst_v, buf, acc_sh):
        c = lax.axis_index("c")
        s = lax.axis_index("s")
        t = c * NS + s
        base = s * ROWS_PT
        pltpu.sync_copy(zacc_hbm, acc_sh.at[pl.ds(base, ROWS_PT)])
        pltpu.sync_copy(src_hbm.at[t], src_v)
        pltpu.sync_copy(dst_hbm.at[t], dst_v)
        plsc.subcore_barrier()

        def step(j, carry):
            pltpu.sync_copy(h_hbm.at[src_v.at[j]], buf)
            pltpu.sync_copy(buf, acc_sh.at[dst_v.at[j]], add=True)
            return carry

        nch = jnp.where(c == 0, NCHUNK0, NCHUNK1)
        lax.fori_loop(0, nch, step, 0)
        plsc.subcore_barrier()
        pltpu.sync_copy(acc_sh.at[pl.ds(base, ROWS_PT)],
                        acc_out.at[c, pl.ds(base, ROWS_PT)])

    return pl.kernel(fn,
                     out_type=jax.ShapeDtypeStruct((NC, N_PAD, D),
                                                   jnp.float32),
                     mesh=_MESH, scratch_types=scratch)


def _make_sc_deg():
    scratch = (
        pltpu.VMEM((NCHUNK, CHUNK), jnp.int32),
        pltpu.VMEM((CHUNK, DEGW), jnp.float32),
        pltpu.VMEM_SHARED((N_PAD, DEGW), jnp.float32),
    )

    def fn(dst_hbm, zdeg_hbm, ones_hbm, deg_out, dst_v, ones_v, deg_sh):
        c = lax.axis_index("c")
        s = lax.axis_index("s")
        t = c * NS + s
        base = s * ROWS_PT
        pltpu.sync_copy(zdeg_hbm, deg_sh.at[pl.ds(base, ROWS_PT)])
        pltpu.sync_copy(ones_hbm, ones_v)
        pltpu.sync_copy(dst_hbm.at[t], dst_v)
        plsc.subcore_barrier()

        def step(j, carry):
            pltpu.sync_copy(ones_v, deg_sh.at[dst_v.at[j]], add=True)
            return carry

        nch = jnp.where(c == 0, NCHUNK0, NCHUNK1)
        lax.fori_loop(0, nch, step, 0)
        plsc.subcore_barrier()
        pltpu.sync_copy(deg_sh.at[pl.ds(base, ROWS_PT)],
                        deg_out.at[c, pl.ds(base, ROWS_PT)])

    return pl.kernel(fn,
                     out_type=jax.ShapeDtypeStruct((NC, N_PAD, DEGW),
                                                   jnp.float32),
                     mesh=_MESH, scratch_types=scratch)


_sc_agg = _make_sc_agg()
_sc_deg = _make_sc_deg()

_TCB = 1000


def _make_tc_layer(ln_relu: bool):
    def body(h_ref, a0_ref, a1_ref, d0_ref, d1_ref, ws_ref, wn_ref,
             b_ref, g_ref, be_ref, o_ref):
        agg = a0_ref[0] + a1_ref[0]
        deg = d0_ref[0][:, :1] + d1_ref[0][:, :1]
        mean = agg / jnp.maximum(deg, 1.0)
        z = (jnp.dot(h_ref[...], ws_ref[...],
                     preferred_element_type=jnp.float32)
             + jnp.dot(mean, wn_ref[...],
                       preferred_element_type=jnp.float32)
             + b_ref[...])
        if ln_relu:
            mu = jnp.mean(z, axis=-1, keepdims=True)
            var = jnp.mean((z - mu) ** 2, axis=-1, keepdims=True)
            z = (z - mu) * lax.rsqrt(var + 1e-5) * g_ref[...] + be_ref[...]
            z = jnp.maximum(z, 0.0)
        o_ref[...] = z

    grid = (N_NODES // _TCB,)
    full = pl.BlockSpec((1, D), lambda i: (0, 0))
    return pl.pallas_call(
        body,
        grid=grid,
        in_specs=[
            pl.BlockSpec((_TCB, D), lambda i: (i, 0)),
            pl.BlockSpec((1, _TCB, D), lambda i: (0, i, 0)),
            pl.BlockSpec((1, _TCB, D), lambda i: (1, i, 0)),
            pl.BlockSpec((1, _TCB, DEGW), lambda i: (0, i, 0)),
            pl.BlockSpec((1, _TCB, DEGW), lambda i: (1, i, 0)),
            pl.BlockSpec((D, D), lambda i: (0, 0)),
            pl.BlockSpec((D, D), lambda i: (0, 0)),
            full, full, full,
        ],
        out_specs=pl.BlockSpec((_TCB, D), lambda i: (i, 0)),
        out_shape=jax.ShapeDtypeStruct((N_NODES, D), jnp.float32),
    )


_tc_layer_lnrelu = _make_tc_layer(True)
_tc_layer_plain = _make_tc_layer(False)


def kernel(x, edge_index, W_self0, W_neigh0, b0, g0, be0,
           W_self1, W_neigh1, b1, g1, be1, W_self2, W_neigh2, b2):
    pad = E_PAD - E_EDGES

    def split_edges(v, padval):
        v = jnp.concatenate([v, jnp.full((pad,), padval, jnp.int32)])
        a = v[:NS * EPT0].reshape(NS, NCHUNK0, CHUNK)
        b = v[NS * EPT0:].reshape(NS, NCHUNK1, CHUNK)
        a = jnp.pad(a, ((0, 0), (0, NCHUNK - NCHUNK0), (0, 0)),
                    constant_values=padval)
        b = jnp.pad(b, ((0, 0), (0, NCHUNK - NCHUNK1), (0, 0)),
                    constant_values=padval)
        return jnp.concatenate([a, b], axis=0)

    src = split_edges(edge_index[0], 0)
    dst = split_edges(edge_index[1], DUMMY_DST)

    zacc = jnp.zeros((ROWS_PT, D), jnp.float32)
    zdeg = jnp.zeros((ROWS_PT, DEGW), jnp.float32)
    ones_c = jnp.ones((CHUNK, DEGW), jnp.float32)

    def pad_w(w):
        return jnp.pad(w, ((0, 0), (0, D - N_CLS)))

    ws2 = pad_w(W_self2)
    wn2 = pad_w(W_neigh2)
    b2p = jnp.pad(b2, (0, D - N_CLS))

    r2 = lambda v: v.reshape(1, D)

    deg = _sc_deg(dst, zdeg, ones_c)
    acc0 = _sc_agg(x, src, dst, zacc)
    h1 = _tc_layer_lnrelu(x, acc0, acc0, deg, deg, W_self0, W_neigh0,
                          r2(b0), r2(g0), r2(be0))
    acc1 = _sc_agg(h1, src, dst, zacc)
    h2 = _tc_layer_lnrelu(h1, acc1, acc1, deg, deg, W_self1, W_neigh1,
                          r2(b1), r2(g1), r2(be1))
    acc2 = _sc_agg(h2, src, dst, zacc)
    out = _tc_layer_plain(h2, acc2, acc2, deg, deg, ws2, wn2,
                          r2(b2p), r2(b2p), r2(b2p))
    return out[:, :N_CLS]


# speedup vs baseline: 1.9361x; 1.1124x over previous
import jax
import jax.numpy as jnp
from jax import lax
from jax.experimental import pallas as pl
from jax.experimental.pallas import tpu as pltpu
from jax.experimental.pallas import tpu_sc as plsc

N_NODES = 10000
E_EDGES = 320000
D = 128
N_CLS = 47

NC = 2
NS = 16
NW = NC * NS
CHUNK = 128
NCHUNK0 = 60
NCHUNK1 = 97
NCHUNK = max(NCHUNK0, NCHUNK1)
EPT0 = NCHUNK0 * CHUNK
EPT1 = NCHUNK1 * CHUNK
E_PAD = NS * (EPT0 + EPT1)
ROWS_PT = 640
N_PAD = NS * ROWS_PT
DUMMY_DST = N_NODES + 16
DEGW = 128


_MESH = plsc.VectorSubcoreMesh(core_axis_name="c", subcore_axis_name="s")


def _make_sc_agg():
    scratch = (
        pltpu.VMEM((NCHUNK, CHUNK), jnp.int32),
        pltpu.VMEM((NCHUNK, CHUNK), jnp.int32),
        pltpu.VMEM((CHUNK, D), jnp.float32),
        pltpu.VMEM_SHARED((N_PAD, D), jnp.float32),
    )

    def fn(h_hbm, src_hbm, dst_hbm, zacc_hbm,
           acc_out, src_v, dst_v, buf, acc_sh):
        c = lax.axis_index("c")
        s = lax.axis_index("s")
        t = c * NS + s
        base = s * ROWS_PT
        pltpu.sync_copy(zacc_hbm, acc_sh.at[pl.ds(base, ROWS_PT)])
        pltpu.sync_copy(src_hbm.at[t], src_v)
        pltpu.sync_copy(dst_hbm.at[t], dst_v)
        plsc.subcore_barrier()

        def step(j, carry):
            pltpu.sync_copy(h_hbm.at[src_v.at[j]], buf)
            pltpu.sync_copy(buf, acc_sh.at[dst_v.at[j]], add=True)
            return carry

        nch = jnp.where(c == 0, NCHUNK0, NCHUNK1)
        lax.fori_loop(0, nch, step, 0)
        plsc.subcore_barrier()
        pltpu.sync_copy(acc_sh.at[pl.ds(base, ROWS_PT)],
                        acc_out.at[c, pl.ds(base, ROWS_PT)])

    return pl.kernel(fn,
                     out_type=jax.ShapeDtypeStruct((NC, N_PAD, D),
                                                   jnp.float32),
                     mesh=_MESH, scratch_types=scratch)


def _make_sc_deg():
    scratch = (
        pltpu.VMEM((NCHUNK, CHUNK), jnp.int32),
        pltpu.VMEM((CHUNK, DEGW), jnp.float32),
        pltpu.VMEM_SHARED((N_PAD, DEGW), jnp.float32),
    )

    def fn(dst_hbm, zdeg_hbm, ones_hbm, deg_out, dst_v, ones_v, deg_sh):
        c = lax.axis_index("c")
        s = lax.axis_index("s")
        t = c * NS + s
        base = s * ROWS_PT
        pltpu.sync_copy(zdeg_hbm, deg_sh.at[pl.ds(base, ROWS_PT)])
        pltpu.sync_copy(ones_hbm, ones_v)
        pltpu.sync_copy(dst_hbm.at[t], dst_v)
        plsc.subcore_barrier()

        def step(j, carry):
            pltpu.sync_copy(ones_v, deg_sh.at[dst_v.at[j]], add=True)
            return carry

        nch = jnp.where(c == 0, NCHUNK0, NCHUNK1)
        lax.fori_loop(0, nch, step, 0)
        plsc.subcore_barrier()
        pltpu.sync_copy(deg_sh.at[pl.ds(base, ROWS_PT)],
                        deg_out.at[c, pl.ds(base, ROWS_PT)])

    return pl.kernel(fn,
                     out_type=jax.ShapeDtypeStruct((NC, N_PAD, DEGW),
                                                   jnp.float32),
                     mesh=_MESH, scratch_types=scratch)


_sc_agg = _make_sc_agg()
_sc_deg = _make_sc_deg()

_TCB = 1000


def _make_tc_layer(ln_relu: bool):
    def body(h_ref, a0_ref, a1_ref, d0_ref, d1_ref, ws_ref, wn_ref,
             b_ref, g_ref, be_ref, o_ref):
        agg = a0_ref[0] + a1_ref[0]
        deg = d0_ref[0][:, :1] + d1_ref[0][:, :1]
        mean = agg / jnp.maximum(deg, 1.0)
        z = (jnp.dot(h_ref[...], ws_ref[...],
                     preferred_element_type=jnp.float32)
             + jnp.dot(mean, wn_ref[...],
                       preferred_element_type=jnp.float32)
             + b_ref[...])
        if ln_relu:
            mu = jnp.mean(z, axis=-1, keepdims=True)
            var = jnp.mean((z - mu) ** 2, axis=-1, keepdims=True)
            z = (z - mu) * lax.rsqrt(var + 1e-5) * g_ref[...] + be_ref[...]
            z = jnp.maximum(z, 0.0)
        o_ref[...] = z

    grid = (N_NODES // _TCB,)
    full = pl.BlockSpec((1, D), lambda i: (0, 0))
    return pl.pallas_call(
        body,
        grid=grid,
        in_specs=[
            pl.BlockSpec((_TCB, D), lambda i: (i, 0)),
            pl.BlockSpec((1, _TCB, D), lambda i: (0, i, 0)),
            pl.BlockSpec((1, _TCB, D), lambda i: (1, i, 0)),
            pl.BlockSpec((1, _TCB, DEGW), lambda i: (0, i, 0)),
            pl.BlockSpec((1, _TCB, DEGW), lambda i: (1, i, 0)),
            pl.BlockSpec((D, D), lambda i: (0, 0)),
            pl.BlockSpec((D, D), lambda i: (0, 0)),
            full, full, full,
        ],
        out_specs=pl.BlockSpec((_TCB, D), lambda i: (i, 0)),
        out_shape=jax.ShapeDtypeStruct((N_NODES, D), jnp.float32),
    )


_tc_layer_lnrelu = _make_tc_layer(True)
_tc_layer_plain = _make_tc_layer(False)


def kernel(x, edge_index, W_self0, W_neigh0, b0, g0, be0,
           W_self1, W_neigh1, b1, g1, be1, W_self2, W_neigh2, b2):
    pad = E_PAD - E_EDGES

    def split_edges(v, padval):
        v = jnp.concatenate([v, jnp.full((pad,), padval, jnp.int32)])
        a = v[:NS * EPT0].reshape(NS, NCHUNK0, CHUNK)
        b = v[NS * EPT0:].reshape(NS, NCHUNK1, CHUNK)
        a = jnp.pad(a, ((0, 0), (0, NCHUNK - NCHUNK0), (0, 0)),
                    constant_values=padval)
        b = jnp.pad(b, ((0, 0), (0, NCHUNK - NCHUNK1), (0, 0)),
                    constant_values=padval)
        return jnp.concatenate([a, b], axis=0)

    src = split_edges(edge_index[0], 0)
    dst = split_edges(edge_index[1], DUMMY_DST)

    zacc = jnp.zeros((ROWS_PT, D), jnp.float32)
    zdeg = jnp.zeros((ROWS_PT, DEGW), jnp.float32)
    ones_c = jnp.ones((CHUNK, DEGW), jnp.float32)

    def pad_w(w):
        return jnp.pad(w, ((0, 0), (0, D - N_CLS)))

    ws2 = pad_w(W_self2)
    wn2 = pad_w(W_neigh2)
    b2p = jnp.pad(b2, (0, D - N_CLS))

    r2 = lambda v: v.reshape(1, D)

    deg = _sc_deg(dst, zdeg, ones_c)
    acc0 = _sc_agg(x, src, dst, zacc)
    h1 = _tc_layer_lnrelu(x, acc0, acc0, deg, deg, W_self0, W_neigh0,
                          r2(b0), r2(g0), r2(be0))
    acc1 = _sc_agg(h1, src, dst, zacc)
    h2 = _tc_layer_lnrelu(h1, acc1, acc1, deg, deg, W_self1, W_neigh1,
                          r2(b1), r2(g1), r2(be1))
    acc2 = _sc_agg(h2, src, dst, zacc)
    out = _tc_layer_plain(h2, acc2, acc2, deg, deg, ws2, wn2,
                          r2(b2p), r2(b2p), r2(b2p))
    return out[:, :N_CLS]


# speedup vs baseline: 2.2326x; 1.1532x over previous
import jax
import jax.numpy as jnp
from jax import lax
from jax.experimental import pallas as pl
from jax.experimental.pallas import tpu as pltpu
from jax.experimental.pallas import tpu_sc as plsc

N_NODES = 10000
E_EDGES = 320000
D = 128
N_CLS = 47

NC = 2
NS = 16
NW = NC * NS
CHUNK = 128
NCHUNK0 = 78
NCHUNK1 = 79
NCHUNK = max(NCHUNK0, NCHUNK1)
EPT0 = NCHUNK0 * CHUNK
EPT1 = NCHUNK1 * CHUNK
E_PAD = NS * (EPT0 + EPT1)
ROWS_PT = 640
N_PAD = NS * ROWS_PT
DUMMY_DST = N_NODES + 16
DEGW = 128


_MESH = plsc.VectorSubcoreMesh(core_axis_name="c", subcore_axis_name="s")


def _make_sc_agg():
    scratch = (
        pltpu.VMEM((NCHUNK, CHUNK), jnp.int32),
        pltpu.VMEM((NCHUNK, CHUNK), jnp.int32),
        pltpu.VMEM((CHUNK, D), jnp.float32),
        pltpu.VMEM_SHARED((N_PAD, D), jnp.float32),
    )

    def fn(h_hbm, src_hbm, dst_hbm, zacc_hbm,
           acc_out, src_v, dst_v, buf, acc_sh):
        c = lax.axis_index("c")
        s = lax.axis_index("s")
        t = c * NS + s
        base = s * ROWS_PT
        pltpu.sync_copy(zacc_hbm, acc_sh.at[pl.ds(base, ROWS_PT)])
        pltpu.sync_copy(src_hbm.at[t], src_v)
        pltpu.sync_copy(dst_hbm.at[t], dst_v)
        plsc.subcore_barrier()

        def step(j, carry):
            pltpu.sync_copy(h_hbm.at[src_v.at[j]], buf)
            pltpu.sync_copy(buf, acc_sh.at[dst_v.at[j]], add=True)
            return carry

        nch = jnp.where(c == 0, NCHUNK0, NCHUNK1)
        lax.fori_loop(0, nch, step, 0)
        plsc.subcore_barrier()
        pltpu.sync_copy(acc_sh.at[pl.ds(base, ROWS_PT)],
                        acc_out.at[c, pl.ds(base, ROWS_PT)])

    return pl.kernel(fn,
                     out_type=jax.ShapeDtypeStruct((NC, N_PAD, D),
                                                   jnp.float32),
                     mesh=_MESH, scratch_types=scratch)


def _make_sc_deg():
    scratch = (
        pltpu.VMEM((NCHUNK, CHUNK), jnp.int32),
        pltpu.VMEM((CHUNK, DEGW), jnp.float32),
        pltpu.VMEM_SHARED((N_PAD, DEGW), jnp.float32),
    )

    def fn(dst_hbm, zdeg_hbm, ones_hbm, deg_out, dst_v, ones_v, deg_sh):
        c = lax.axis_index("c")
        s = lax.axis_index("s")
        t = c * NS + s
        base = s * ROWS_PT
        pltpu.sync_copy(zdeg_hbm, deg_sh.at[pl.ds(base, ROWS_PT)])
        pltpu.sync_copy(ones_hbm, ones_v)
        pltpu.sync_copy(dst_hbm.at[t], dst_v)
        plsc.subcore_barrier()

        def step(j, carry):
            pltpu.sync_copy(ones_v, deg_sh.at[dst_v.at[j]], add=True)
            return carry

        nch = jnp.where(c == 0, NCHUNK0, NCHUNK1)
        lax.fori_loop(0, nch, step, 0)
        plsc.subcore_barrier()
        pltpu.sync_copy(deg_sh.at[pl.ds(base, ROWS_PT)],
                        deg_out.at[c, pl.ds(base, ROWS_PT)])

    return pl.kernel(fn,
                     out_type=jax.ShapeDtypeStruct((NC, N_PAD, DEGW),
                                                   jnp.float32),
                     mesh=_MESH, scratch_types=scratch)


_sc_agg = _make_sc_agg()
_sc_deg = _make_sc_deg()

_TCB = 1000


def _make_tc_layer(ln_relu: bool):
    def body(h_ref, a0_ref, a1_ref, d0_ref, d1_ref, ws_ref, wn_ref,
             b_ref, g_ref, be_ref, o_ref):
        agg = a0_ref[0] + a1_ref[0]
        deg = d0_ref[0][:, :1] + d1_ref[0][:, :1]
        mean = agg / jnp.maximum(deg, 1.0)
        z = (jnp.dot(h_ref[...], ws_ref[...],
                     preferred_element_type=jnp.float32)
             + jnp.dot(mean, wn_ref[...],
                       preferred_element_type=jnp.float32)
             + b_ref[...])
        if ln_relu:
            mu = jnp.mean(z, axis=-1, keepdims=True)
            var = jnp.mean((z - mu) ** 2, axis=-1, keepdims=True)
            z = (z - mu) * lax.rsqrt(var + 1e-5) * g_ref[...] + be_ref[...]
            z = jnp.maximum(z, 0.0)
        o_ref[...] = z

    grid = (N_NODES // _TCB,)
    full = pl.BlockSpec((1, D), lambda i: (0, 0))
    return pl.pallas_call(
        body,
        grid=grid,
        in_specs=[
            pl.BlockSpec((_TCB, D), lambda i: (i, 0)),
            pl.BlockSpec((1, _TCB, D), lambda i: (0, i, 0)),
            pl.BlockSpec((1, _TCB, D), lambda i: (1, i, 0)),
            pl.BlockSpec((1, _TCB, DEGW), lambda i: (0, i, 0)),
            pl.BlockSpec((1, _TCB, DEGW), lambda i: (1, i, 0)),
            pl.BlockSpec((D, D), lambda i: (0, 0)),
            pl.BlockSpec((D, D), lambda i: (0, 0)),
            full, full, full,
        ],
        out_specs=pl.BlockSpec((_TCB, D), lambda i: (i, 0)),
        out_shape=jax.ShapeDtypeStruct((N_NODES, D), jnp.float32),
    )


_tc_layer_lnrelu = _make_tc_layer(True)
_tc_layer_plain = _make_tc_layer(False)


def kernel(x, edge_index, W_self0, W_neigh0, b0, g0, be0,
           W_self1, W_neigh1, b1, g1, be1, W_self2, W_neigh2, b2):
    pad = E_PAD - E_EDGES

    def split_edges(v, padval):
        v = jnp.concatenate([v, jnp.full((pad,), padval, jnp.int32)])
        a = v[:NS * EPT0].reshape(NS, NCHUNK0, CHUNK)
        b = v[NS * EPT0:].reshape(NS, NCHUNK1, CHUNK)
        a = jnp.pad(a, ((0, 0), (0, NCHUNK - NCHUNK0), (0, 0)),
                    constant_values=padval)
        b = jnp.pad(b, ((0, 0), (0, NCHUNK - NCHUNK1), (0, 0)),
                    constant_values=padval)
        return jnp.concatenate([a, b], axis=0)

    src = split_edges(edge_index[0], 0)
    dst = split_edges(edge_index[1], DUMMY_DST)

    zacc = jnp.zeros((ROWS_PT, D), jnp.float32)
    zdeg = jnp.zeros((ROWS_PT, DEGW), jnp.float32)
    ones_c = jnp.ones((CHUNK, DEGW), jnp.float32)

    def pad_w(w):
        return jnp.pad(w, ((0, 0), (0, D - N_CLS)))

    ws2 = pad_w(W_self2)
    wn2 = pad_w(W_neigh2)
    b2p = jnp.pad(b2, (0, D - N_CLS))

    r2 = lambda v: v.reshape(1, D)

    deg = _sc_deg(dst, zdeg, ones_c)
    acc0 = _sc_agg(x, src, dst, zacc)
    h1 = _tc_layer_lnrelu(x, acc0, acc0, deg, deg, W_self0, W_neigh0,
                          r2(b0), r2(g0), r2(be0))
    acc1 = _sc_agg(h1, src, dst, zacc)
    h2 = _tc_layer_lnrelu(h1, acc1, acc1, deg, deg, W_self1, W_neigh1,
                          r2(b1), r2(g1), r2(be1))
    acc2 = _sc_agg(h2, src, dst, zacc)
    out = _tc_layer_plain(h2, acc2, acc2, deg, deg, ws2, wn2,
                          r2(b2p), r2(b2p), r2(b2p))
    return out[:, :N_CLS]
